# Initial kernel scaffold; baseline (speedup 1.0000x reference)
import jax
import jax.numpy as jnp
from jax.experimental import pallas as pl


def kernel(x_entropy_p16, x_entropy_p8):
    raise NotImplementedError("write your pallas kernel here")


# speedup vs baseline: 3.0247x; 3.0247x over previous
import jax
import jax.numpy as jnp
from jax import lax
from jax.experimental import pallas as pl
from jax.experimental.pallas import tpu as pltpu

COARSE = 0.3
MEDIUM = 0.3
N16 = 64 * 32 * 32
N8 = 64 * 64 * 64
K_COARSE = round(N16 * COARSE)
K_MED = round(4 * N16 * COARSE + N8 * MEDIUM)

def _ordered_bits(x):
    b = lax.bitcast_convert_type(x, jnp.int32)
    return b ^ (lax.shift_right_arithmetic(b, 31) & 2147483647)


def _bits_to_f32(o):
    b = o ^ (lax.shift_right_arithmetic(o, 31) & 2147483647)
    return lax.bitcast_convert_type(b, jnp.float32)


def _kth_smallest(o_ref, k):
    cnt_neg = jnp.sum((o_ref[...] < 0).astype(jnp.int32))
    base = jnp.where(k <= cnt_neg, jnp.int32(-2147483648), jnp.int32(0))

    def body(i, prefix):
        b = jnp.int32(30) - i
        low = lax.shift_left(jnp.int32(1), b) - 1
        test = base + prefix + low
        cnt = jnp.sum((o_ref[...] <= test).astype(jnp.int32))
        bit = jnp.where(cnt >= k, jnp.int32(0), lax.shift_left(jnp.int32(1), b))
        return prefix + bit

    prefix = lax.fori_loop(0, 31, body, jnp.int32(0))
    return base + prefix


def _thr_body(p16_ref, p8_ref, p16u2_ref, thr_ref, o16_ref, o8_ref):
    o16_ref[...] = _ordered_bits(p16_ref[...])
    t16b = _kth_smallest(o16_ref, K_COARSE)
    thr16 = _bits_to_f32(t16b)

    gc_up = (p16u2_ref[...] < thr16).astype(jnp.float32)
    p8m = p8_ref[...] * (1.0 - gc_up)
    o8_ref[...] = _ordered_bits(p8m)
    t8b = _kth_smallest(o8_ref, K_MED)
    thr8 = _bits_to_f32(t8b)

    row = lax.broadcasted_iota(jnp.int32, (8, 128), 0)
    col = lax.broadcasted_iota(jnp.int32, (8, 128), 1)
    out = jnp.where((row == 0) & (col == 0), thr16,
                    jnp.where((row == 0) & (col == 1), thr8, 0.0))
    thr_ref[...] = out


def _rep_mats(out_n, in_n):
    s = out_n // in_n
    r_out = lax.broadcasted_iota(jnp.int32, (out_n, in_n), 0) // s
    r_in = lax.broadcasted_iota(jnp.int32, (out_n, in_n), 1)
    rmat = (r_out == r_in).astype(jnp.float32)
    c_in = lax.broadcasted_iota(jnp.int32, (in_n, out_n), 0)
    c_out = lax.broadcasted_iota(jnp.int32, (in_n, out_n), 1) // s
    cmat = (c_in == c_out).astype(jnp.float32)
    return rmat, cmat


def _dot(a, b):
    return jnp.dot(a, b, preferred_element_type=jnp.float32)


def _gate_body(thr_ref, p16_ref, p8_ref, m0_ref, m1_ref, m2_ref, gate_ref):
    t16 = thr_ref[0, 0]
    t8 = thr_ref[0, 1]
    p16 = p16_ref[0]
    p8 = p8_ref[0]

    gc = p16 < t16
    gcf = gc.astype(jnp.float32)
    m0_ref[0, 0] = gc.astype(jnp.int32)

    r2a, c2a = _rep_mats(64, 32)
    u2 = _dot(_dot(r2a, gcf), c2a)
    gm = (p8 < t8) & (u2 == 0.0)
    gmf = gm.astype(jnp.float32)
    m1_ref[0, 0] = gm.astype(jnp.int32)

    r4, c4 = _rep_mats(128, 32)
    cf = _dot(_dot(r4, gcf), c4)
    r2b, c2b = _rep_mats(128, 64)
    mf = _dot(_dot(r2b, gmf), c2b)
    ff = 1.0 - cf - mf
    m2_ref[0, 0] = (ff != 0.0).astype(jnp.int32)
    gate_ref[0, 0, :, 0:128] = cf
    gate_ref[0, 0, :, 128:256] = mf
    gate_ref[0, 0, :, 256:384] = ff


def _make_thr_call(interpret=False):
    return pl.pallas_call(
        _thr_body,
        out_shape=jax.ShapeDtypeStruct((8, 128), jnp.float32),
        scratch_shapes=[
            pltpu.VMEM((512, 128), jnp.int32),
            pltpu.VMEM((2048, 128), jnp.int32),
        ],
        interpret=interpret,
    )


def _make_gate_call(interpret=False):
    return pl.pallas_call(
        _gate_body,
        grid=(64,),
        in_specs=[
            pl.BlockSpec((8, 128), lambda b: (0, 0)),
            pl.BlockSpec((1, 32, 32), lambda b: (b, 0, 0)),
            pl.BlockSpec((1, 64, 64), lambda b: (b, 0, 0)),
        ],
        out_specs=[
            pl.BlockSpec((1, 1, 32, 32), lambda b: (b, 0, 0, 0)),
            pl.BlockSpec((1, 1, 64, 64), lambda b: (b, 0, 0, 0)),
            pl.BlockSpec((1, 1, 128, 128), lambda b: (b, 0, 0, 0)),
            pl.BlockSpec((1, 1, 128, 384), lambda b: (b, 0, 0, 0)),
        ],
        out_shape=[
            jax.ShapeDtypeStruct((64, 1, 32, 32), jnp.int32),
            jax.ShapeDtypeStruct((64, 1, 64, 64), jnp.int32),
            jax.ShapeDtypeStruct((64, 1, 128, 128), jnp.int32),
            jax.ShapeDtypeStruct((64, 1, 128, 384), jnp.float32),
        ],
        interpret=interpret,
    )


def _kernel_impl(x_entropy_p16, x_entropy_p8, interpret=False):
    p16f = x_entropy_p16.reshape(512, 128)
    p8f = x_entropy_p8.reshape(2048, 128)
    p16u2 = jnp.repeat(jnp.repeat(x_entropy_p16, 2, axis=1), 2, axis=2)
    p16u2f = p16u2.reshape(2048, 128)

    thr = _make_thr_call(interpret)(p16f, p8f, p16u2f)
    m0, m1, m2, gate = _make_gate_call(interpret)(thr, x_entropy_p16, x_entropy_p8)
    return m0, m1, m2, gate


@jax.jit
def kernel(x_entropy_p16, x_entropy_p8):
    return _kernel_impl(x_entropy_p16, x_entropy_p8)
